# baseline (speedup 1.0000x reference)
import functools

import jax
import jax.numpy as jnp
from jax import lax
from jax.experimental import pallas as pl
from jax.experimental.pallas import tpu as pltpu
from jax.experimental.pallas import tpu_sc as plsc

BATCH = 16384
EMBED = 64
STYLE = 128
N_EMO = 32
N_SPK = 100000

NC = 2
NS = 16
NW = NC * NS
B_PER_W = BATCH // NW
IDX_CHUNK = 128
N_CHUNKS = B_PER_W // IDX_CHUNK
LANES = 16

PB = 1024
N_BLK = (BATCH // 2) // PB


def _sc_gather_body(idx_hbm, table_hbm, out_hbm, idx_v, rows_v, pairs_v,
                    gsem, osem):
    wid = lax.axis_index("s") * NC + lax.axis_index("c")
    pltpu.sync_copy(idx_hbm.at[wid], idx_v)
    copies = [
        pltpu.async_copy(
            table_hbm.at[idx_v.at[pl.ds(j * IDX_CHUNK, IDX_CHUNK)]],
            rows_v.at[pl.ds(j * IDX_CHUNK, IDX_CHUNK)],
            gsem,
        )
        for j in range(N_CHUNKS)
    ]
    for j in range(N_CHUNKS):
        copies[j].wait()
        group_idx = []
        for g in range(IDX_CHUNK // LANES):
            kvec = lax.iota(jnp.int32, LANES) + (j * IDX_CHUNK + g * LANES)
            prow = lax.shift_right_logical(kvec, 1)
            pcol0 = (kvec & 1) * EMBED
            group_idx.append((kvec, prow, pcol0))

        def col_body(c, carry):
            cvec = jnp.zeros((LANES,), jnp.int32) + c
            for kvec, prow, pcol0 in group_idx:
                val = plsc.load_gather(rows_v, [kvec, cvec])
                plsc.store_scatter(pairs_v, [prow, pcol0 + cvec], val)
            return carry

        lax.fori_loop(0, EMBED, col_body, 0)
    pltpu.async_copy(
        pairs_v, out_hbm.at[pl.ds(wid * (B_PER_W // 2), B_PER_W // 2)], osem
    ).wait()


@functools.lru_cache(maxsize=None)
def _make_spk_gather():
    return pl.kernel(
        _sc_gather_body,
        out_type=jax.ShapeDtypeStruct((BATCH // 2, 2 * EMBED), jnp.float32),
        mesh=plsc.VectorSubcoreMesh(core_axis_name="c", subcore_axis_name="s",
                                    num_cores=NC, num_subcores=NS),
        scratch_types=[
            pltpu.VMEM((B_PER_W,), jnp.int32),
            pltpu.VMEM((B_PER_W, EMBED), jnp.float32),
            pltpu.VMEM((B_PER_W // 2, 2 * EMBED), jnp.float32),
            pltpu.SemaphoreType.DMA,
            pltpu.SemaphoreType.DMA,
        ],
        compiler_params=pltpu.CompilerParams(use_tc_tiling_on_sc=False,
                                             needs_layout_passes=False),
    )


def _mlp_body(pairs_ref, eide_ref, eido_ref, emo_ref, w1s_ref, w1e_ref,
              b1_ref, w2_ref, b2_ref, out_ref):
    pairs = pairs_ref[...]
    emo_proj = jnp.dot(emo_ref[...], w1e_ref[...],
                       preferred_element_type=jnp.float32) + b1_ref[...]
    iot = lax.broadcasted_iota(jnp.int32, (PB, N_EMO), 1)

    def half(spk, eid):
        onehot = (eid[:, None] == iot).astype(jnp.float32)
        h = (jnp.dot(spk, w1s_ref[...], preferred_element_type=jnp.float32)
             + jnp.dot(onehot, emo_proj, preferred_element_type=jnp.float32))
        h = jnp.maximum(h, 0.0)
        return jnp.dot(h, w2_ref[...],
                       preferred_element_type=jnp.float32) + b2_ref[...]

    out_e = half(pairs[:, :EMBED], eide_ref[0, 0, :])
    out_o = half(pairs[:, EMBED:], eido_ref[0, 0, :])
    both = jnp.concatenate([out_e[:, None, :], out_o[:, None, :]], axis=1)
    out_ref[...] = both.reshape(2 * PB, STYLE)


def _mlp(pairs, eide3, eido3, emo_table, w1sT, w1eT, b1, w2T, b2):
    return pl.pallas_call(
        _mlp_body,
        grid=(N_BLK,),
        in_specs=[
            pl.BlockSpec((PB, 2 * EMBED), lambda i: (i, 0)),
            pl.BlockSpec((1, 1, PB), lambda i: (i, 0, 0)),
            pl.BlockSpec((1, 1, PB), lambda i: (i, 0, 0)),
            pl.BlockSpec((N_EMO, EMBED), lambda i: (0, 0)),
            pl.BlockSpec((EMBED, STYLE), lambda i: (0, 0)),
            pl.BlockSpec((EMBED, STYLE), lambda i: (0, 0)),
            pl.BlockSpec((1, STYLE), lambda i: (0, 0)),
            pl.BlockSpec((STYLE, STYLE), lambda i: (0, 0)),
            pl.BlockSpec((1, STYLE), lambda i: (0, 0)),
        ],
        out_specs=pl.BlockSpec((2 * PB, STYLE), lambda i: (i, 0)),
        out_shape=jax.ShapeDtypeStruct((BATCH, STYLE), jnp.float32),
    )(pairs, eide3, eido3, emo_table, w1sT, w1eT, b1, w2T, b2)


def kernel(speaker_id, emotion_id, spk_table, emo_table, W1, b1, W2, b2):
    idx = speaker_id.astype(jnp.int32).reshape(NW, B_PER_W)
    pairs = _make_spk_gather()(idx, spk_table)
    eid = emotion_id.astype(jnp.int32).reshape(BATCH // 2, 2)
    eide3 = eid[:, 0].reshape(N_BLK, 1, PB)
    eido3 = eid[:, 1].reshape(N_BLK, 1, PB)
    w1sT = W1[:, :EMBED].T
    w1eT = W1[:, EMBED:].T
    out = _mlp(pairs, eide3, eido3, emo_table, w1sT, w1eT,
               b1.reshape(1, STYLE), W2.T, b2.reshape(1, STYLE))
    return out


# speedup vs baseline: 1.2844x; 1.2844x over previous
import functools

import jax
import jax.numpy as jnp
from jax import lax
from jax.experimental import pallas as pl
from jax.experimental.pallas import tpu as pltpu
from jax.experimental.pallas import tpu_sc as plsc

BATCH = 16384
EMBED = 64
STYLE = 128
N_EMO = 32
N_SPK = 100000

NC = 2
NS = 16
NW = NC * NS
B_PER_W = BATCH // NW
IDX_CHUNK = 128
N_CHUNKS = B_PER_W // IDX_CHUNK
LANES = 16

PB = 1024
N_BLK = (BATCH // 2) // PB


def _sc_gather_body(idx_hbm, table_hbm, out_hbm, idx_v, rows_v, pairs_v,
                    gsem, osem):
    wid = lax.axis_index("s") * NC + lax.axis_index("c")
    pltpu.sync_copy(idx_hbm.at[wid], idx_v)
    copies = [
        pltpu.async_copy(
            table_hbm.at[idx_v.at[pl.ds(j * IDX_CHUNK, IDX_CHUNK)]],
            rows_v.at[pl.ds(j * IDX_CHUNK, IDX_CHUNK)],
            gsem,
        )
        for j in range(N_CHUNKS)
    ]
    for j in range(N_CHUNKS):
        copies[j].wait()

        def pair_body(p, carry):
            for parity in range(2):
                k = 2 * p + parity
                for c in range(EMBED // LANES):
                    pairs_v[p, pl.ds(parity * EMBED + c * LANES, LANES)] = (
                        rows_v[k, pl.ds(c * LANES, LANES)])
            return carry

        lax.fori_loop(j * (IDX_CHUNK // 2), (j + 1) * (IDX_CHUNK // 2),
                      pair_body, 0)
    pltpu.async_copy(
        pairs_v, out_hbm.at[pl.ds(wid * (B_PER_W // 2), B_PER_W // 2)], osem
    ).wait()


@functools.lru_cache(maxsize=None)
def _make_spk_gather():
    return pl.kernel(
        _sc_gather_body,
        out_type=jax.ShapeDtypeStruct((BATCH // 2, 2 * EMBED), jnp.float32),
        mesh=plsc.VectorSubcoreMesh(core_axis_name="c", subcore_axis_name="s",
                                    num_cores=NC, num_subcores=NS),
        scratch_types=[
            pltpu.VMEM((B_PER_W,), jnp.int32),
            pltpu.VMEM((B_PER_W, EMBED), jnp.float32),
            pltpu.VMEM((B_PER_W // 2, 2 * EMBED), jnp.float32),
            pltpu.SemaphoreType.DMA,
            pltpu.SemaphoreType.DMA,
        ],
        compiler_params=pltpu.CompilerParams(use_tc_tiling_on_sc=False,
                                             needs_layout_passes=False),
    )


def _mlp_body(pairs_ref, eide_ref, eido_ref, emo_ref, w1s_ref, w1e_ref,
              b1_ref, w2_ref, b2_ref, out_ref):
    pairs = pairs_ref[...]
    emo_proj = jnp.dot(emo_ref[...], w1e_ref[...],
                       preferred_element_type=jnp.float32) + b1_ref[...]
    iot = lax.broadcasted_iota(jnp.int32, (PB, N_EMO), 1)

    def half(spk, eid):
        onehot = (eid[:, None] == iot).astype(jnp.float32)
        h = (jnp.dot(spk, w1s_ref[...], preferred_element_type=jnp.float32)
             + jnp.dot(onehot, emo_proj, preferred_element_type=jnp.float32))
        h = jnp.maximum(h, 0.0)
        return jnp.dot(h, w2_ref[...],
                       preferred_element_type=jnp.float32) + b2_ref[...]

    out_e = half(pairs[:, :EMBED], eide_ref[0, 0, :])
    out_o = half(pairs[:, EMBED:], eido_ref[0, 0, :])
    both = jnp.concatenate([out_e[:, None, :], out_o[:, None, :]], axis=1)
    out_ref[...] = both.reshape(2 * PB, STYLE)


def _mlp(pairs, eide3, eido3, emo_table, w1sT, w1eT, b1, w2T, b2):
    return pl.pallas_call(
        _mlp_body,
        grid=(N_BLK,),
        in_specs=[
            pl.BlockSpec((PB, 2 * EMBED), lambda i: (i, 0)),
            pl.BlockSpec((1, 1, PB), lambda i: (i, 0, 0)),
            pl.BlockSpec((1, 1, PB), lambda i: (i, 0, 0)),
            pl.BlockSpec((N_EMO, EMBED), lambda i: (0, 0)),
            pl.BlockSpec((EMBED, STYLE), lambda i: (0, 0)),
            pl.BlockSpec((EMBED, STYLE), lambda i: (0, 0)),
            pl.BlockSpec((1, STYLE), lambda i: (0, 0)),
            pl.BlockSpec((STYLE, STYLE), lambda i: (0, 0)),
            pl.BlockSpec((1, STYLE), lambda i: (0, 0)),
        ],
        out_specs=pl.BlockSpec((2 * PB, STYLE), lambda i: (i, 0)),
        out_shape=jax.ShapeDtypeStruct((BATCH, STYLE), jnp.float32),
    )(pairs, eide3, eido3, emo_table, w1sT, w1eT, b1, w2T, b2)


def kernel(speaker_id, emotion_id, spk_table, emo_table, W1, b1, W2, b2):
    idx = speaker_id.astype(jnp.int32).reshape(NW, B_PER_W)
    pairs = _make_spk_gather()(idx, spk_table)
    eid = emotion_id.astype(jnp.int32).reshape(BATCH // 2, 2)
    eide3 = eid[:, 0].reshape(N_BLK, 1, PB)
    eido3 = eid[:, 1].reshape(N_BLK, 1, PB)
    w1sT = W1[:, :EMBED].T
    w1eT = W1[:, EMBED:].T
    out = _mlp(pairs, eide3, eido3, emo_table, w1sT, w1eT,
               b1.reshape(1, STYLE), W2.T, b2.reshape(1, STYLE))
    return out


# speedup vs baseline: 1.3433x; 1.0458x over previous
import functools

import jax
import jax.numpy as jnp
from jax import lax
from jax.experimental import pallas as pl
from jax.experimental.pallas import tpu as pltpu
from jax.experimental.pallas import tpu_sc as plsc

BATCH = 16384
EMBED = 64
STYLE = 128
N_EMO = 32
N_SPK = 100000

NC = 2
NS = 16
NW = NC * NS
B_PER_W = BATCH // NW
IDX_CHUNK = 128
N_CHUNKS = B_PER_W // IDX_CHUNK
LANES = 16

BB = 2048
N_BLK = BATCH // BB


def _sc_gather_body(idx_hbm, table_hbm, out_hbm, idx_v, idxh_v, rows_v,
                    gsem, osem):
    wid = lax.axis_index("s") * NC + lax.axis_index("c")
    base = wid * B_PER_W
    pltpu.sync_copy(idx_hbm.at[pl.ds(base, B_PER_W)], idx_v)
    for g in range(B_PER_W // LANES):
        sl = pl.ds(g * LANES, LANES)
        idxh_v[sl] = lax.shift_right_logical(idx_v[sl], 1)
    copies = [
        pltpu.async_copy(
            table_hbm.at[idxh_v.at[pl.ds(j * IDX_CHUNK, IDX_CHUNK)]],
            rows_v.at[pl.ds(j * IDX_CHUNK, IDX_CHUNK)],
            gsem,
        )
        for j in range(N_CHUNKS)
    ]
    for c in copies:
        c.wait()
    pltpu.async_copy(rows_v, out_hbm.at[pl.ds(base, B_PER_W)], osem).wait()


@functools.lru_cache(maxsize=None)
def _make_spk_gather():
    return pl.kernel(
        _sc_gather_body,
        out_type=jax.ShapeDtypeStruct((BATCH, 2 * EMBED), jnp.float32),
        mesh=plsc.VectorSubcoreMesh(core_axis_name="c", subcore_axis_name="s",
                                    num_cores=NC, num_subcores=NS),
        scratch_types=[
            pltpu.VMEM((B_PER_W,), jnp.int32),
            pltpu.VMEM((B_PER_W,), jnp.int32),
            pltpu.VMEM((B_PER_W, 2 * EMBED), jnp.float32),
            pltpu.SemaphoreType.DMA,
            pltpu.SemaphoreType.DMA,
        ],
        compiler_params=pltpu.CompilerParams(use_tc_tiling_on_sc=False),
    )


def _mlp_body(pairs_ref, sid_ref, eid_ref, emo_ref, w1s_ref, w1e_ref,
              b1_ref, w2_ref, b2_ref, out_ref):
    sid = sid_ref[...]
    eid = eid_ref[...]
    pairs = pairs_ref[...]
    odd = (sid & 1)[:, None] == jnp.ones((1, EMBED), jnp.int32)
    spk = jnp.where(odd, pairs[:, EMBED:], pairs[:, :EMBED])
    onehot = (eid[:, None] == lax.broadcasted_iota(jnp.int32, (BB, N_EMO), 1)
              ).astype(jnp.float32)
    emo_proj = jnp.dot(emo_ref[...], w1e_ref[...],
                       preferred_element_type=jnp.float32) + b1_ref[...]
    h = (jnp.dot(spk, w1s_ref[...], preferred_element_type=jnp.float32)
         + jnp.dot(onehot, emo_proj, preferred_element_type=jnp.float32))
    h = jnp.maximum(h, 0.0)
    out_ref[...] = jnp.dot(h, w2_ref[...],
                           preferred_element_type=jnp.float32) + b2_ref[...]


def _mlp(pairs, sid, eid, emo_table, w1sT, w1eT, b1, w2T, b2):
    return pl.pallas_call(
        _mlp_body,
        grid=(N_BLK,),
        in_specs=[
            pl.BlockSpec((BB, 2 * EMBED), lambda i: (i, 0)),
            pl.BlockSpec((BB,), lambda i: (i,)),
            pl.BlockSpec((BB,), lambda i: (i,)),
            pl.BlockSpec((N_EMO, EMBED), lambda i: (0, 0)),
            pl.BlockSpec((EMBED, STYLE), lambda i: (0, 0)),
            pl.BlockSpec((EMBED, STYLE), lambda i: (0, 0)),
            pl.BlockSpec((1, STYLE), lambda i: (0, 0)),
            pl.BlockSpec((STYLE, STYLE), lambda i: (0, 0)),
            pl.BlockSpec((1, STYLE), lambda i: (0, 0)),
        ],
        out_specs=pl.BlockSpec((BB, STYLE), lambda i: (i, 0)),
        out_shape=jax.ShapeDtypeStruct((BATCH, STYLE), jnp.float32),
    )(pairs, sid, eid, emo_table, w1sT, w1eT, b1, w2T, b2)


def kernel(speaker_id, emotion_id, spk_table, emo_table, W1, b1, W2, b2):
    sid = speaker_id.astype(jnp.int32)
    eid = emotion_id.astype(jnp.int32)
    pairs = _make_spk_gather()(sid, spk_table.reshape(N_SPK // 2, 2 * EMBED))
    w1sT = W1[:, :EMBED].T
    w1eT = W1[:, EMBED:].T
    out = _mlp(pairs, sid, eid, emo_table, w1sT, w1eT,
               b1.reshape(1, STYLE), W2.T, b2.reshape(1, STYLE))
    return out


# speedup vs baseline: 1.6878x; 1.2565x over previous
import functools

import jax
import jax.numpy as jnp
from jax import lax
from jax.experimental import pallas as pl
from jax.experimental.pallas import tpu as pltpu
from jax.experimental.pallas import tpu_sc as plsc

BATCH = 16384
EMBED = 64
STYLE = 128
N_EMO = 32
N_SPK = 100000

NC = 2
NS = 16
NW = NC * NS
B_PER_W = BATCH // NW
IDX_CHUNK = 128
N_CHUNKS = B_PER_W // IDX_CHUNK
LANES = 16

BB = 2048
N_BLK = BATCH // BB

TB = 12800
N_TBLK = -(-N_SPK // TB)


def _transpose_body(tt_ref, out_ref):
    blk = tt_ref[...]
    tr = jnp.swapaxes(blk, 0, 1)
    tr3 = tr.reshape(TB // 2, 2, EMBED)
    out_ref[...] = jnp.concatenate([tr3[:, 0, :], tr3[:, 1, :]], axis=-1)


def _pair_table(tableT):
    return pl.pallas_call(
        _transpose_body,
        grid=(N_TBLK,),
        in_specs=[pl.BlockSpec((EMBED, TB), lambda i: (0, i))],
        out_specs=pl.BlockSpec((TB // 2, 2 * EMBED), lambda i: (i, 0)),
        out_shape=jax.ShapeDtypeStruct((N_SPK // 2, 2 * EMBED), jnp.float32),
    )(tableT)


def _sc_gather_body(idx_hbm, table_hbm, out_hbm, idx_v, idxh_v, rows_v,
                    gsem, osem):
    wid = lax.axis_index("s") * NC + lax.axis_index("c")
    base = wid * B_PER_W
    pltpu.sync_copy(idx_hbm.at[pl.ds(base, B_PER_W)], idx_v)
    for g in range(B_PER_W // LANES):
        sl = pl.ds(g * LANES, LANES)
        idxh_v[sl] = lax.shift_right_logical(idx_v[sl], 1)
    copies = [
        pltpu.async_copy(
            table_hbm.at[idxh_v.at[pl.ds(j * IDX_CHUNK, IDX_CHUNK)]],
            rows_v.at[pl.ds(j * IDX_CHUNK, IDX_CHUNK)],
            gsem,
        )
        for j in range(N_CHUNKS)
    ]
    for c in copies:
        c.wait()
    pltpu.async_copy(rows_v, out_hbm.at[pl.ds(base, B_PER_W)], osem).wait()


@functools.lru_cache(maxsize=None)
def _make_spk_gather():
    return pl.kernel(
        _sc_gather_body,
        out_type=jax.ShapeDtypeStruct((BATCH, 2 * EMBED), jnp.float32),
        mesh=plsc.VectorSubcoreMesh(core_axis_name="c", subcore_axis_name="s",
                                    num_cores=NC, num_subcores=NS),
        scratch_types=[
            pltpu.VMEM((B_PER_W,), jnp.int32),
            pltpu.VMEM((B_PER_W,), jnp.int32),
            pltpu.VMEM((B_PER_W, 2 * EMBED), jnp.float32),
            pltpu.SemaphoreType.DMA,
            pltpu.SemaphoreType.DMA,
        ],
        compiler_params=pltpu.CompilerParams(use_tc_tiling_on_sc=False),
    )


def _mlp_body(pairs_ref, sid_ref, eid_ref, emo_ref, w1s_ref, w1e_ref,
              b1_ref, w2_ref, b2_ref, out_ref):
    sid = sid_ref[...]
    eid = eid_ref[...]
    pairs = pairs_ref[...]
    odd = (sid & 1)[:, None] == jnp.ones((1, EMBED), jnp.int32)
    spk = jnp.where(odd, pairs[:, EMBED:], pairs[:, :EMBED])
    onehot = (eid[:, None] == lax.broadcasted_iota(jnp.int32, (BB, N_EMO), 1)
              ).astype(jnp.float32)
    emo_proj = jnp.dot(emo_ref[...], w1e_ref[...],
                       preferred_element_type=jnp.float32) + b1_ref[...]
    h = (jnp.dot(spk, w1s_ref[...], preferred_element_type=jnp.float32)
         + jnp.dot(onehot, emo_proj, preferred_element_type=jnp.float32))
    h = jnp.maximum(h, 0.0)
    out_ref[...] = jnp.dot(h, w2_ref[...],
                           preferred_element_type=jnp.float32) + b2_ref[...]


def _mlp(pairs, sid, eid, emo_table, w1sT, w1eT, b1, w2T, b2):
    return pl.pallas_call(
        _mlp_body,
        grid=(N_BLK,),
        in_specs=[
            pl.BlockSpec((BB, 2 * EMBED), lambda i: (i, 0)),
            pl.BlockSpec((BB,), lambda i: (i,)),
            pl.BlockSpec((BB,), lambda i: (i,)),
            pl.BlockSpec((N_EMO, EMBED), lambda i: (0, 0)),
            pl.BlockSpec((EMBED, STYLE), lambda i: (0, 0)),
            pl.BlockSpec((EMBED, STYLE), lambda i: (0, 0)),
            pl.BlockSpec((1, STYLE), lambda i: (0, 0)),
            pl.BlockSpec((STYLE, STYLE), lambda i: (0, 0)),
            pl.BlockSpec((1, STYLE), lambda i: (0, 0)),
        ],
        out_specs=pl.BlockSpec((BB, STYLE), lambda i: (i, 0)),
        out_shape=jax.ShapeDtypeStruct((BATCH, STYLE), jnp.float32),
    )(pairs, sid, eid, emo_table, w1sT, w1eT, b1, w2T, b2)


def kernel(speaker_id, emotion_id, spk_table, emo_table, W1, b1, W2, b2):
    sid = speaker_id.astype(jnp.int32)
    eid = emotion_id.astype(jnp.int32)
    pairs = _make_spk_gather()(sid, _pair_table(spk_table.T))
    w1sT = W1[:, :EMBED].T
    w1eT = W1[:, EMBED:].T
    out = _mlp(pairs, sid, eid, emo_table, w1sT, w1eT,
               b1.reshape(1, STYLE), W2.T, b2.reshape(1, STYLE))
    return out


# speedup vs baseline: 2.1444x; 1.2705x over previous
import functools

import jax
import jax.numpy as jnp
from jax import lax
from jax.experimental import pallas as pl
from jax.experimental.pallas import tpu as pltpu
from jax.experimental.pallas import tpu_sc as plsc

BATCH = 16384
EMBED = 64
STYLE = 128
N_EMO = 32
N_SPK = 100000

NC = 2
NS = 16
NW = NC * NS
B_PER_W = BATCH // NW
IDX_CHUNK = 128
N_CHUNKS = B_PER_W // IDX_CHUNK

BB = 2048
N_BLK = BATCH // BB

TB = 12800
N_TBLK = -(-N_SPK // TB)


def _project_body(tt_ref, w1s_ref, out_ref):
    out_ref[...] = lax.dot_general(
        tt_ref[...], w1s_ref[...], (((0,), (0,)), ((), ())),
        preferred_element_type=jnp.float32)


def _project_table(tableT, w1sT):
    return pl.pallas_call(
        _project_body,
        grid=(N_TBLK,),
        in_specs=[
            pl.BlockSpec((EMBED, TB), lambda i: (0, i)),
            pl.BlockSpec((EMBED, STYLE), lambda i: (0, 0)),
        ],
        out_specs=pl.BlockSpec((TB, STYLE), lambda i: (i, 0)),
        out_shape=jax.ShapeDtypeStruct((N_TBLK * TB, STYLE), jnp.float32),
    )(tableT, w1sT)


def _sc_gather_body(idx_hbm, table_hbm, out_hbm, idx_v, rows_v, gsem, osem):
    wid = lax.axis_index("s") * NC + lax.axis_index("c")
    base = wid * B_PER_W
    pltpu.sync_copy(idx_hbm.at[pl.ds(base, B_PER_W)], idx_v)
    copies = [
        pltpu.async_copy(
            table_hbm.at[idx_v.at[pl.ds(j * IDX_CHUNK, IDX_CHUNK)]],
            rows_v.at[pl.ds(j * IDX_CHUNK, IDX_CHUNK)],
            gsem,
        )
        for j in range(N_CHUNKS)
    ]
    for c in copies:
        c.wait()
    pltpu.async_copy(rows_v, out_hbm.at[pl.ds(base, B_PER_W)], osem).wait()


@functools.lru_cache(maxsize=None)
def _make_spk_gather():
    return pl.kernel(
        _sc_gather_body,
        out_type=jax.ShapeDtypeStruct((BATCH, STYLE), jnp.float32),
        mesh=plsc.VectorSubcoreMesh(core_axis_name="c", subcore_axis_name="s",
                                    num_cores=NC, num_subcores=NS),
        scratch_types=[
            pltpu.VMEM((B_PER_W,), jnp.int32),
            pltpu.VMEM((B_PER_W, STYLE), jnp.float32),
            pltpu.SemaphoreType.DMA,
            pltpu.SemaphoreType.DMA,
        ],
        compiler_params=pltpu.CompilerParams(use_tc_tiling_on_sc=False),
    )


def _mlp_body(spk_ref, eid_ref, emo_ref, w1e_ref, b1_ref, w2_ref, b2_ref,
              out_ref):
    eid = eid_ref[...]
    onehot = (eid[:, None] == lax.broadcasted_iota(jnp.int32, (BB, N_EMO), 1)
              ).astype(jnp.float32)
    emo_proj = jnp.dot(emo_ref[...], w1e_ref[...],
                       preferred_element_type=jnp.float32) + b1_ref[...]
    h = spk_ref[...] + jnp.dot(onehot, emo_proj,
                               preferred_element_type=jnp.float32)
    h = jnp.maximum(h, 0.0)
    out_ref[...] = jnp.dot(h, w2_ref[...],
                           preferred_element_type=jnp.float32) + b2_ref[...]


def _mlp(spk, eid, emo_table, w1eT, b1, w2T, b2):
    return pl.pallas_call(
        _mlp_body,
        grid=(N_BLK,),
        in_specs=[
            pl.BlockSpec((BB, STYLE), lambda i: (i, 0)),
            pl.BlockSpec((BB,), lambda i: (i,)),
            pl.BlockSpec((N_EMO, EMBED), lambda i: (0, 0)),
            pl.BlockSpec((EMBED, STYLE), lambda i: (0, 0)),
            pl.BlockSpec((1, STYLE), lambda i: (0, 0)),
            pl.BlockSpec((STYLE, STYLE), lambda i: (0, 0)),
            pl.BlockSpec((1, STYLE), lambda i: (0, 0)),
        ],
        out_specs=pl.BlockSpec((BB, STYLE), lambda i: (i, 0)),
        out_shape=jax.ShapeDtypeStruct((BATCH, STYLE), jnp.float32),
    )(spk, eid, emo_table, w1eT, b1, w2T, b2)


def kernel(speaker_id, emotion_id, spk_table, emo_table, W1, b1, W2, b2):
    sid = speaker_id.astype(jnp.int32)
    eid = emotion_id.astype(jnp.int32)
    w1sT = W1[:, :EMBED].T
    w1eT = W1[:, EMBED:].T
    proj = _project_table(spk_table.T, w1sT)
    spk = _make_spk_gather()(sid, proj)
    out = _mlp(spk, eid, emo_table, w1eT,
               b1.reshape(1, STYLE), W2.T, b2.reshape(1, STYLE))
    return out


# speedup vs baseline: 2.2461x; 1.0474x over previous
import functools

import jax
import jax.numpy as jnp
from jax import lax
from jax.experimental import pallas as pl
from jax.experimental.pallas import tpu as pltpu
from jax.experimental.pallas import tpu_sc as plsc

BATCH = 16384
EMBED = 64
STYLE = 128
N_EMO = 32
N_SPK = 100000

NC = 2
NS = 16
NW = NC * NS
B_PER_W = BATCH // NW
IDX_CHUNK = 128
N_CHUNKS = B_PER_W // IDX_CHUNK

BB = 4096
N_BLK = BATCH // BB

TB = 12800
N_TBLK = -(-N_SPK // TB)


def _project_body(tt_ref, w1s_ref, out_ref):
    out_ref[...] = lax.dot_general(
        tt_ref[...], w1s_ref[...], (((0,), (0,)), ((), ())),
        preferred_element_type=jnp.float32)


def _project_table(tableT, w1sT):
    return pl.pallas_call(
        _project_body,
        grid=(N_TBLK,),
        in_specs=[
            pl.BlockSpec((EMBED, TB), lambda i: (0, i)),
            pl.BlockSpec((EMBED, STYLE), lambda i: (0, 0)),
        ],
        out_specs=pl.BlockSpec((TB, STYLE), lambda i: (i, 0)),
        out_shape=jax.ShapeDtypeStruct((N_TBLK * TB, STYLE), jnp.float32),
    )(tableT, w1sT)


def _sc_gather_body(idx_hbm, table_hbm, out_hbm, idx_v, rows_v, gsem, osem):
    wid = lax.axis_index("s") * NC + lax.axis_index("c")
    base = wid * B_PER_W
    pltpu.sync_copy(idx_hbm.at[pl.ds(base, B_PER_W)], idx_v)
    copies = [
        pltpu.async_copy(
            table_hbm.at[idx_v.at[pl.ds(j * IDX_CHUNK, IDX_CHUNK)]],
            rows_v.at[pl.ds(j * IDX_CHUNK, IDX_CHUNK)],
            gsem,
        )
        for j in range(N_CHUNKS)
    ]
    for c in copies:
        c.wait()
    pltpu.async_copy(rows_v, out_hbm.at[pl.ds(base, B_PER_W)], osem).wait()


@functools.lru_cache(maxsize=None)
def _make_spk_gather():
    return pl.kernel(
        _sc_gather_body,
        out_type=jax.ShapeDtypeStruct((BATCH, STYLE), jnp.float32),
        mesh=plsc.VectorSubcoreMesh(core_axis_name="c", subcore_axis_name="s",
                                    num_cores=NC, num_subcores=NS),
        scratch_types=[
            pltpu.VMEM((B_PER_W,), jnp.int32),
            pltpu.VMEM((B_PER_W, STYLE), jnp.float32),
            pltpu.SemaphoreType.DMA,
            pltpu.SemaphoreType.DMA,
        ],
        compiler_params=pltpu.CompilerParams(use_tc_tiling_on_sc=False),
    )


def _mlp_body(spk_ref, eid_ref, emo_ref, w1e_ref, b1_ref, w2_ref, b2_ref,
              out_ref):
    eid = eid_ref[...]
    onehot = (eid[:, None] == lax.broadcasted_iota(jnp.int32, (BB, N_EMO), 1)
              ).astype(jnp.float32)
    emo_proj = jnp.dot(emo_ref[...], w1e_ref[...],
                       preferred_element_type=jnp.float32) + b1_ref[...]
    h = spk_ref[...] + jnp.dot(onehot, emo_proj,
                               preferred_element_type=jnp.float32)
    h = jnp.maximum(h, 0.0)
    out_ref[...] = jnp.dot(h, w2_ref[...],
                           preferred_element_type=jnp.float32) + b2_ref[...]


def _mlp(spk, eid, emo_table, w1eT, b1, w2T, b2):
    return pl.pallas_call(
        _mlp_body,
        grid=(N_BLK,),
        in_specs=[
            pl.BlockSpec((BB, STYLE), lambda i: (i, 0)),
            pl.BlockSpec((BB,), lambda i: (i,)),
            pl.BlockSpec((N_EMO, EMBED), lambda i: (0, 0)),
            pl.BlockSpec((EMBED, STYLE), lambda i: (0, 0)),
            pl.BlockSpec((1, STYLE), lambda i: (0, 0)),
            pl.BlockSpec((STYLE, STYLE), lambda i: (0, 0)),
            pl.BlockSpec((1, STYLE), lambda i: (0, 0)),
        ],
        out_specs=pl.BlockSpec((BB, STYLE), lambda i: (i, 0)),
        out_shape=jax.ShapeDtypeStruct((BATCH, STYLE), jnp.float32),
    )(spk, eid, emo_table, w1eT, b1, w2T, b2)


def kernel(speaker_id, emotion_id, spk_table, emo_table, W1, b1, W2, b2):
    sid = speaker_id.astype(jnp.int32)
    eid = emotion_id.astype(jnp.int32)
    w1sT = W1[:, :EMBED].T
    w1eT = W1[:, EMBED:].T
    proj = _project_table(spk_table.T, w1sT)
    spk = _make_spk_gather()(sid, proj)
    out = _mlp(spk, eid, emo_table, w1eT,
               b1.reshape(1, STYLE), W2.T, b2.reshape(1, STYLE))
    return out


# speedup vs baseline: 2.2703x; 1.0108x over previous
import functools

import jax
import jax.numpy as jnp
from jax import lax
from jax.experimental import pallas as pl
from jax.experimental.pallas import tpu as pltpu
from jax.experimental.pallas import tpu_sc as plsc

BATCH = 16384
EMBED = 64
STYLE = 128
N_EMO = 32
N_SPK = 100000

NC = 2
NS = 16
NW = NC * NS
B_PER_W = BATCH // NW
IDX_CHUNK = 128
N_CHUNKS = B_PER_W // IDX_CHUNK

BB = 4096
N_BLK = BATCH // BB

TB = 25600
N_TBLK = -(-N_SPK // TB)


def _project_body(tt_ref, w1s_ref, out_ref):
    out_ref[...] = lax.dot_general(
        tt_ref[...], w1s_ref[...], (((0,), (0,)), ((), ())),
        preferred_element_type=jnp.float32)


def _project_table(tableT, w1sT):
    return pl.pallas_call(
        _project_body,
        grid=(N_TBLK,),
        in_specs=[
            pl.BlockSpec((EMBED, TB), lambda i: (0, i)),
            pl.BlockSpec((EMBED, STYLE), lambda i: (0, 0)),
        ],
        out_specs=pl.BlockSpec((TB, STYLE), lambda i: (i, 0)),
        out_shape=jax.ShapeDtypeStruct((N_TBLK * TB, STYLE), jnp.float32),
        compiler_params=pltpu.CompilerParams(
            vmem_limit_bytes=56 * 1024 * 1024),
    )(tableT, w1sT)


def _sc_gather_body(idx_hbm, table_hbm, out_hbm, idx_v, rows_v, gsem, osem):
    wid = lax.axis_index("s") * NC + lax.axis_index("c")
    base = wid * B_PER_W
    pltpu.sync_copy(idx_hbm.at[pl.ds(base, B_PER_W)], idx_v)
    copies = [
        pltpu.async_copy(
            table_hbm.at[idx_v.at[pl.ds(j * IDX_CHUNK, IDX_CHUNK)]],
            rows_v.at[pl.ds(j * IDX_CHUNK, IDX_CHUNK)],
            gsem,
        )
        for j in range(N_CHUNKS)
    ]
    for c in copies:
        c.wait()
    pltpu.async_copy(rows_v, out_hbm.at[pl.ds(base, B_PER_W)], osem).wait()


@functools.lru_cache(maxsize=None)
def _make_spk_gather():
    return pl.kernel(
        _sc_gather_body,
        out_type=jax.ShapeDtypeStruct((BATCH, STYLE), jnp.float32),
        mesh=plsc.VectorSubcoreMesh(core_axis_name="c", subcore_axis_name="s",
                                    num_cores=NC, num_subcores=NS),
        scratch_types=[
            pltpu.VMEM((B_PER_W,), jnp.int32),
            pltpu.VMEM((B_PER_W, STYLE), jnp.float32),
            pltpu.SemaphoreType.DMA,
            pltpu.SemaphoreType.DMA,
        ],
        compiler_params=pltpu.CompilerParams(use_tc_tiling_on_sc=False),
    )


def _mlp_body(spk_ref, eid_ref, emo_ref, w1e_ref, b1_ref, w2_ref, b2_ref,
              out_ref):
    eid = eid_ref[...]
    onehot = (eid[:, None] == lax.broadcasted_iota(jnp.int32, (BB, N_EMO), 1)
              ).astype(jnp.float32)
    emo_proj = jnp.dot(emo_ref[...], w1e_ref[...],
                       preferred_element_type=jnp.float32) + b1_ref[...]
    h = spk_ref[...] + jnp.dot(onehot, emo_proj,
                               preferred_element_type=jnp.float32)
    h = jnp.maximum(h, 0.0)
    out_ref[...] = jnp.dot(h, w2_ref[...],
                           preferred_element_type=jnp.float32) + b2_ref[...]


def _mlp(spk, eid, emo_table, w1eT, b1, w2T, b2):
    return pl.pallas_call(
        _mlp_body,
        grid=(N_BLK,),
        in_specs=[
            pl.BlockSpec((BB, STYLE), lambda i: (i, 0)),
            pl.BlockSpec((BB,), lambda i: (i,)),
            pl.BlockSpec((N_EMO, EMBED), lambda i: (0, 0)),
            pl.BlockSpec((EMBED, STYLE), lambda i: (0, 0)),
            pl.BlockSpec((1, STYLE), lambda i: (0, 0)),
            pl.BlockSpec((STYLE, STYLE), lambda i: (0, 0)),
            pl.BlockSpec((1, STYLE), lambda i: (0, 0)),
        ],
        out_specs=pl.BlockSpec((BB, STYLE), lambda i: (i, 0)),
        out_shape=jax.ShapeDtypeStruct((BATCH, STYLE), jnp.float32),
    )(spk, eid, emo_table, w1eT, b1, w2T, b2)


def kernel(speaker_id, emotion_id, spk_table, emo_table, W1, b1, W2, b2):
    sid = speaker_id.astype(jnp.int32)
    eid = emotion_id.astype(jnp.int32)
    w1sT = W1[:, :EMBED].T
    w1eT = W1[:, EMBED:].T
    proj = _project_table(spk_table.T, w1sT)
    spk = _make_spk_gather()(sid, proj)
    out = _mlp(spk, eid, emo_table, w1eT,
               b1.reshape(1, STYLE), W2.T, b2.reshape(1, STYLE))
    return out


# speedup vs baseline: 2.3017x; 1.0138x over previous
import functools

import jax
import jax.numpy as jnp
from jax import lax
from jax.experimental import pallas as pl
from jax.experimental.pallas import tpu as pltpu
from jax.experimental.pallas import tpu_sc as plsc

BATCH = 16384
EMBED = 64
STYLE = 128
N_EMO = 32
N_SPK = 100000

NC = 2
NS = 16
NW = NC * NS
B_PER_W = BATCH // NW
IDX_CHUNK = 128
N_CHUNKS = B_PER_W // IDX_CHUNK

BB = 8192
N_BLK = BATCH // BB

TB = 25600
N_TBLK = -(-N_SPK // TB)


def _project_body(tt_ref, w1s_ref, out_ref):
    out_ref[...] = lax.dot_general(
        tt_ref[...], w1s_ref[...], (((0,), (0,)), ((), ())),
        preferred_element_type=jnp.float32)


def _project_table(tableT, w1sT):
    return pl.pallas_call(
        _project_body,
        grid=(N_TBLK,),
        in_specs=[
            pl.BlockSpec((EMBED, TB), lambda i: (0, i)),
            pl.BlockSpec((EMBED, STYLE), lambda i: (0, 0)),
        ],
        out_specs=pl.BlockSpec((TB, STYLE), lambda i: (i, 0)),
        out_shape=jax.ShapeDtypeStruct((N_TBLK * TB, STYLE), jnp.float32),
        compiler_params=pltpu.CompilerParams(
            vmem_limit_bytes=56 * 1024 * 1024),
    )(tableT, w1sT)


def _sc_gather_body(idx_hbm, table_hbm, out_hbm, idx_v, rows_v, gsem, osem):
    wid = lax.axis_index("s") * NC + lax.axis_index("c")
    base = wid * B_PER_W
    pltpu.sync_copy(idx_hbm.at[pl.ds(base, B_PER_W)], idx_v)
    copies = [
        pltpu.async_copy(
            table_hbm.at[idx_v.at[pl.ds(j * IDX_CHUNK, IDX_CHUNK)]],
            rows_v.at[pl.ds(j * IDX_CHUNK, IDX_CHUNK)],
            gsem,
        )
        for j in range(N_CHUNKS)
    ]
    outs = []
    for j in range(N_CHUNKS):
        copies[j].wait()
        sl = pl.ds(j * IDX_CHUNK, IDX_CHUNK)
        outs.append(
            pltpu.async_copy(
                rows_v.at[sl], out_hbm.at[pl.ds(base + j * IDX_CHUNK,
                                                IDX_CHUNK)], osem))
    for o in outs:
        o.wait()


@functools.lru_cache(maxsize=None)
def _make_spk_gather():
    return pl.kernel(
        _sc_gather_body,
        out_type=jax.ShapeDtypeStruct((BATCH, STYLE), jnp.float32),
        mesh=plsc.VectorSubcoreMesh(core_axis_name="c", subcore_axis_name="s",
                                    num_cores=NC, num_subcores=NS),
        scratch_types=[
            pltpu.VMEM((B_PER_W,), jnp.int32),
            pltpu.VMEM((B_PER_W, STYLE), jnp.float32),
            pltpu.SemaphoreType.DMA,
            pltpu.SemaphoreType.DMA,
        ],
        compiler_params=pltpu.CompilerParams(use_tc_tiling_on_sc=False),
    )


def _mlp_body(spk_ref, eid_ref, emo_ref, w1e_ref, b1_ref, w2_ref, b2_ref,
              out_ref):
    eid = eid_ref[...]
    onehot = (eid[:, None] == lax.broadcasted_iota(jnp.int32, (BB, N_EMO), 1)
              ).astype(jnp.float32)
    emo_proj = jnp.dot(emo_ref[...], w1e_ref[...],
                       preferred_element_type=jnp.float32) + b1_ref[...]
    h = spk_ref[...] + jnp.dot(onehot, emo_proj,
                               preferred_element_type=jnp.float32)
    h = jnp.maximum(h, 0.0)
    out_ref[...] = jnp.dot(h, w2_ref[...],
                           preferred_element_type=jnp.float32) + b2_ref[...]


def _mlp(spk, eid, emo_table, w1eT, b1, w2T, b2):
    return pl.pallas_call(
        _mlp_body,
        grid=(N_BLK,),
        in_specs=[
            pl.BlockSpec((BB, STYLE), lambda i: (i, 0)),
            pl.BlockSpec((BB,), lambda i: (i,)),
            pl.BlockSpec((N_EMO, EMBED), lambda i: (0, 0)),
            pl.BlockSpec((EMBED, STYLE), lambda i: (0, 0)),
            pl.BlockSpec((1, STYLE), lambda i: (0, 0)),
            pl.BlockSpec((STYLE, STYLE), lambda i: (0, 0)),
            pl.BlockSpec((1, STYLE), lambda i: (0, 0)),
        ],
        out_specs=pl.BlockSpec((BB, STYLE), lambda i: (i, 0)),
        out_shape=jax.ShapeDtypeStruct((BATCH, STYLE), jnp.float32),
    )(spk, eid, emo_table, w1eT, b1, w2T, b2)


def kernel(speaker_id, emotion_id, spk_table, emo_table, W1, b1, W2, b2):
    sid = speaker_id.astype(jnp.int32)
    eid = emotion_id.astype(jnp.int32)
    w1sT = W1[:, :EMBED].T
    w1eT = W1[:, EMBED:].T
    proj = _project_table(spk_table.T, w1sT)
    spk = _make_spk_gather()(sid, proj)
    out = _mlp(spk, eid, emo_table, w1eT,
               b1.reshape(1, STYLE), W2.T, b2.reshape(1, STYLE))
    return out
